# baseline (speedup 1.0000x reference)
import jax
import jax.numpy as jnp
from jax.experimental import pallas as pl
from jax.experimental.pallas import tpu as pltpu

H_IN = 28
H_C1 = 26
H_C2 = 24
H_P = 12
C1 = 32
C2 = 64
N_FEAT = H_P * H_P * C2
N_HID = 128
N_CLS = 10

BB = 128


def _net_kernel(x_ref, m1_ref, b1_ref, m2_ref, b2_ref,
                w1_ref, bf1_ref, w2_ref, bf2_ref, o_ref):
    x = x_ref[...]

    xd = jnp.concatenate(
        [x[:, di:di + H_C1, :].reshape(BB * H_C1, H_IN) for di in range(3)],
        axis=1)
    acc1 = jnp.dot(xd, m1_ref[...], preferred_element_type=jnp.float32)
    c1 = jnp.maximum(acc1 + b1_ref[...], 0.0).astype(jnp.bfloat16)
    c1v = c1.reshape(BB, H_C1, 896)

    e_chunks, o_chunks = [], []
    for wg in range(6):
        acc2 = jnp.zeros((BB * H_C2, 256), jnp.float32)
        for di in range(3):
            cd = c1v[:, di:di + H_C2, 128 * wg:128 * wg + 256]
            acc2 += jnp.dot(cd.reshape(BB * H_C2, 256), m2_ref[di, wg],
                            preferred_element_type=jnp.float32)
        chunk = jnp.maximum(acc2 + b2_ref[:, 256 * wg:256 * wg + 256], 0.0)
        chunk = chunk.astype(jnp.bfloat16)
        e_chunks.append(chunk[:, :128])
        o_chunks.append(chunk[:, 128:])
    c2 = jnp.concatenate(e_chunks + o_chunks, axis=1)

    c2p = c2.reshape(BB * H_P, 2 * H_C2 * C2)
    hp = jnp.maximum(c2p[:, :H_C2 * C2], c2p[:, H_C2 * C2:])
    pooled = jnp.maximum(hp[:, :H_P * C2], hp[:, H_P * C2:])

    feats = pooled.reshape(BB, N_FEAT)
    h = jnp.maximum(
        jnp.dot(feats, w1_ref[...], preferred_element_type=jnp.float32)
        + bf1_ref[...], 0.0).astype(jnp.bfloat16)
    logits = jnp.dot(h, w2_ref[...],
                     preferred_element_type=jnp.float32) + bf2_ref[...]
    m = jnp.max(logits, axis=-1, keepdims=True)
    s = logits - m
    lse = jnp.log(jnp.sum(jnp.exp(s), axis=-1, keepdims=True))
    o_ref[...] = (s - lse).astype(o_ref.dtype)


def kernel(x, m1, b1row, m2, b2row, lsel, rsel, wfc1, bfc1, wfc2, bfc2):
    del lsel, rsel
    B = x.shape[0]
    xb = x.reshape(B, H_IN, H_IN).astype(jnp.bfloat16)
    m1cat = jnp.pad(m1.reshape(3 * H_IN, H_C1 * C1), ((0, 0), (0, 64)))
    b1p = jnp.pad(b1row, ((0, 0), (0, 64)))
    wperm = jnp.arange(24).reshape(6, 2, 2).transpose(0, 2, 1).reshape(24)
    col_idx = (wperm[:, None] * C2 + jnp.arange(C2)[None, :]).reshape(-1)
    m2pad = jnp.pad(m2[:, :, col_idx], ((0, 0), (0, 64), (0, 0)))
    m2p = jnp.stack([
        jnp.stack([m2pad[di, 128 * wg:128 * wg + 256, 256 * wg:256 * wg + 256]
                   for wg in range(6)])
        for di in range(3)])
    b2p = b2row[:, col_idx]
    b_pad = (B + BB - 1) // BB * BB
    if b_pad != B:
        xb = jnp.pad(xb, ((0, b_pad - B), (0, 0), (0, 0)))
    out = pl.pallas_call(
        _net_kernel,
        out_shape=jax.ShapeDtypeStruct((b_pad, N_CLS), jnp.float32),
        grid=(b_pad // BB,),
        in_specs=[
            pl.BlockSpec((BB, H_IN, H_IN), lambda b: (b, 0, 0)),
            pl.BlockSpec((3 * H_IN, 896), lambda b: (0, 0)),
            pl.BlockSpec((1, 896), lambda b: (0, 0)),
            pl.BlockSpec((3, 6, 256, 256), lambda b: (0, 0, 0, 0)),
            pl.BlockSpec((1, H_C2 * C2), lambda b: (0, 0)),
            pl.BlockSpec((N_FEAT, N_HID), lambda b: (0, 0)),
            pl.BlockSpec((1, N_HID), lambda b: (0, 0)),
            pl.BlockSpec((N_HID, N_CLS), lambda b: (0, 0)),
            pl.BlockSpec((1, N_CLS), lambda b: (0, 0)),
        ],
        out_specs=pl.BlockSpec((BB, N_CLS), lambda b: (b, 0)),
        compiler_params=pltpu.CompilerParams(
            dimension_semantics=("parallel",),
            vmem_limit_bytes=56 * 1024 * 1024),
    )(xb, m1cat, b1p, m2p, b2p, wfc1, bfc1, wfc2, bfc2)
    return out[:B]


# speedup vs baseline: 1.2323x; 1.2323x over previous
import jax
import jax.numpy as jnp
from jax.experimental import pallas as pl
from jax.experimental.pallas import tpu as pltpu

H_IN = 28
H_C1 = 26
H_C2 = 24
H_P = 12
C1 = 32
C2 = 64
N_FEAT = H_P * H_P * C2
N_HID = 128
N_CLS = 10

BB = 32


def _net_kernel(x_ref, m1_ref, b1_ref, m2_ref, b2_ref,
                w1_ref, bf1_ref, w2_ref, bf2_ref, o_ref):
    x = x_ref[...]

    xd = jnp.concatenate(
        [x[:, di:di + H_C1, :].reshape(BB * H_C1, H_IN) for di in range(3)],
        axis=1)
    acc1 = jnp.dot(xd, m1_ref[...], preferred_element_type=jnp.float32)
    c1 = jnp.maximum(acc1 + b1_ref[...], 0.0).astype(jnp.bfloat16)
    c1v = c1.reshape(BB, H_C1, 896)

    e_chunks, o_chunks = [], []
    for wg in range(6):
        acc2 = jnp.zeros((BB * H_C2, 256), jnp.float32)
        for di in range(3):
            cd = c1v[:, di:di + H_C2, 128 * wg:128 * wg + 256]
            acc2 += jnp.dot(cd.reshape(BB * H_C2, 256), m2_ref[di, wg],
                            preferred_element_type=jnp.float32)
        chunk = jnp.maximum(acc2 + b2_ref[:, 256 * wg:256 * wg + 256], 0.0)
        chunk = chunk.astype(jnp.bfloat16)
        e_chunks.append(chunk[:, :128])
        o_chunks.append(chunk[:, 128:])
    c2 = jnp.concatenate(e_chunks + o_chunks, axis=1)

    c2p = c2.reshape(BB * H_P, 2 * H_C2 * C2)
    hp = jnp.maximum(c2p[:, :H_C2 * C2], c2p[:, H_C2 * C2:])
    pooled = jnp.maximum(hp[:, :H_P * C2], hp[:, H_P * C2:])

    feats = pooled.reshape(BB, N_FEAT)
    h = jnp.maximum(
        jnp.dot(feats, w1_ref[...], preferred_element_type=jnp.float32)
        + bf1_ref[...], 0.0).astype(jnp.bfloat16)
    logits = jnp.dot(h, w2_ref[...],
                     preferred_element_type=jnp.float32) + bf2_ref[...]
    m = jnp.max(logits, axis=-1, keepdims=True)
    s = logits - m
    lse = jnp.log(jnp.sum(jnp.exp(s), axis=-1, keepdims=True))
    o_ref[...] = (s - lse).astype(o_ref.dtype)


def kernel(x, m1, b1row, m2, b2row, lsel, rsel, wfc1, bfc1, wfc2, bfc2):
    del lsel, rsel
    B = x.shape[0]
    xb = x.reshape(B, H_IN, H_IN).astype(jnp.bfloat16)
    m1cat = jnp.pad(m1.reshape(3 * H_IN, H_C1 * C1), ((0, 0), (0, 64)))
    b1p = jnp.pad(b1row, ((0, 0), (0, 64)))
    wperm = jnp.arange(24).reshape(6, 2, 2).transpose(0, 2, 1).reshape(24)
    col_idx = (wperm[:, None] * C2 + jnp.arange(C2)[None, :]).reshape(-1)
    m2pad = jnp.pad(m2[:, :, col_idx], ((0, 0), (0, 64), (0, 0)))
    m2p = jnp.stack([
        jnp.stack([m2pad[di, 128 * wg:128 * wg + 256, 256 * wg:256 * wg + 256]
                   for wg in range(6)])
        for di in range(3)])
    b2p = b2row[:, col_idx]
    b_pad = (B + BB - 1) // BB * BB
    if b_pad != B:
        xb = jnp.pad(xb, ((0, b_pad - B), (0, 0), (0, 0)))
    out = pl.pallas_call(
        _net_kernel,
        out_shape=jax.ShapeDtypeStruct((b_pad, N_CLS), jnp.float32),
        grid=(b_pad // BB,),
        in_specs=[
            pl.BlockSpec((BB, H_IN, H_IN), lambda b: (b, 0, 0)),
            pl.BlockSpec((3 * H_IN, 896), lambda b: (0, 0)),
            pl.BlockSpec((1, 896), lambda b: (0, 0)),
            pl.BlockSpec((3, 6, 256, 256), lambda b: (0, 0, 0, 0)),
            pl.BlockSpec((1, H_C2 * C2), lambda b: (0, 0)),
            pl.BlockSpec((N_FEAT, N_HID), lambda b: (0, 0)),
            pl.BlockSpec((1, N_HID), lambda b: (0, 0)),
            pl.BlockSpec((N_HID, N_CLS), lambda b: (0, 0)),
            pl.BlockSpec((1, N_CLS), lambda b: (0, 0)),
        ],
        out_specs=pl.BlockSpec((BB, N_CLS), lambda b: (b, 0)),
        compiler_params=pltpu.CompilerParams(
            dimension_semantics=("parallel",),
            vmem_limit_bytes=56 * 1024 * 1024),
    )(xb, m1cat, b1p, m2p, b2p, wfc1, bfc1, wfc2, bfc2)
    return out[:B]


# speedup vs baseline: 2.5087x; 2.0359x over previous
import jax
import jax.numpy as jnp
from jax.experimental import pallas as pl
from jax.experimental.pallas import tpu as pltpu

H_IN = 28
H_C1 = 26
H_C2 = 24
H_P = 12
C1 = 32
C2 = 64
N_FEAT = H_P * H_P * C2
N_HID = 128
N_CLS = 10

BB = 64


def _net_kernel(x_ref, m1_ref, b1_ref, m2_ref, b2_ref,
                w1_ref, bf1_ref, w2_ref, bf2_ref, o_ref):
    xd = x_ref[...].reshape(H_C1 * BB, 3 * H_IN)
    acc1 = jnp.dot(xd, m1_ref[...], preferred_element_type=jnp.float32)
    c1 = jnp.maximum(acc1 + b1_ref[...], 0.0).astype(jnp.bfloat16)
    c1v = c1.reshape(H_C1, BB, 896)

    e_chunks, o_chunks = [], []
    for wg in range(6):
        acc2 = jnp.zeros((H_C2 * BB, 256), jnp.float32)
        for di in range(3):
            cd = c1v[di:di + H_C2, :, 128 * wg:128 * wg + 256]
            acc2 += jnp.dot(cd.reshape(H_C2 * BB, 256), m2_ref[di, wg],
                            preferred_element_type=jnp.float32)
        chunk = jnp.maximum(acc2 + b2_ref[:, 256 * wg:256 * wg + 256], 0.0)
        chunk = chunk.astype(jnp.bfloat16)
        e_chunks.append(chunk[:, :128])
        o_chunks.append(chunk[:, 128:])
    c2 = jnp.concatenate(e_chunks + o_chunks, axis=1)

    c2v = c2.reshape(H_P, 2, BB, H_C2 * C2)
    hp = jnp.maximum(c2v[:, 0], c2v[:, 1]).reshape(H_P * BB, H_C2 * C2)
    pooled = jnp.maximum(hp[:, :H_P * C2], hp[:, H_P * C2:])
    pooled = pooled.astype(jnp.bfloat16)

    u = jnp.dot(pooled, w1_ref[...],
                preferred_element_type=jnp.float32)
    uv = u.reshape(H_P, BB, H_P * N_HID)
    acc = bf1_ref[...]
    for p in range(H_P):
        acc = acc + uv[p, :, 128 * p:128 * p + 128]
    h = jnp.maximum(acc, 0.0).astype(jnp.bfloat16)

    logits = jnp.dot(h, w2_ref[...],
                     preferred_element_type=jnp.float32) + bf2_ref[...]
    m = jnp.max(logits, axis=-1, keepdims=True)
    s = logits - m
    lse = jnp.log(jnp.sum(jnp.exp(s), axis=-1, keepdims=True))
    o_ref[...] = (s - lse).astype(o_ref.dtype)


def kernel(x, m1, b1row, m2, b2row, lsel, rsel, wfc1, bfc1, wfc2, bfc2):
    del lsel, rsel
    B = x.shape[0]
    xb = x.reshape(B, H_IN, H_IN).astype(jnp.bfloat16)
    xcat = jnp.concatenate(
        [xb[:, di:di + H_C1, :] for di in range(3)],
        axis=2).transpose(1, 0, 2)
    m1cat = jnp.pad(m1.reshape(3 * H_IN, H_C1 * C1), ((0, 0), (0, 64)))
    b1p = jnp.pad(b1row, ((0, 0), (0, 64)))
    wperm = jnp.arange(24).reshape(6, 2, 2).transpose(0, 2, 1).reshape(24)
    col_idx = (wperm[:, None] * C2 + jnp.arange(C2)[None, :]).reshape(-1)
    m2pad = jnp.pad(m2[:, :, col_idx], ((0, 0), (0, 64), (0, 0)))
    m2p = jnp.stack([
        jnp.stack([m2pad[di, 128 * wg:128 * wg + 256, 256 * wg:256 * wg + 256]
                   for wg in range(6)])
        for di in range(3)])
    b2p = b2row[:, col_idx]
    wcat = wfc1.reshape(H_P, H_P * C2, N_HID).transpose(1, 0, 2)
    wcat = wcat.reshape(H_P * C2, H_P * N_HID)
    out = pl.pallas_call(
        _net_kernel,
        out_shape=jax.ShapeDtypeStruct((B, N_CLS), jnp.float32),
        grid=(B // BB,),
        in_specs=[
            pl.BlockSpec((H_C1, BB, 3 * H_IN), lambda b: (0, b, 0)),
            pl.BlockSpec((3 * H_IN, 896), lambda b: (0, 0)),
            pl.BlockSpec((1, 896), lambda b: (0, 0)),
            pl.BlockSpec((3, 6, 256, 256), lambda b: (0, 0, 0, 0)),
            pl.BlockSpec((1, H_C2 * C2), lambda b: (0, 0)),
            pl.BlockSpec((H_P * C2, H_P * N_HID), lambda b: (0, 0)),
            pl.BlockSpec((1, N_HID), lambda b: (0, 0)),
            pl.BlockSpec((N_HID, N_CLS), lambda b: (0, 0)),
            pl.BlockSpec((1, N_CLS), lambda b: (0, 0)),
        ],
        out_specs=pl.BlockSpec((BB, N_CLS), lambda b: (b, 0)),
        compiler_params=pltpu.CompilerParams(
            dimension_semantics=("parallel",),
            vmem_limit_bytes=56 * 1024 * 1024),
    )(xcat, m1cat, b1p, m2p, b2p, wcat, bfc1, wfc2, bfc2)
    return out
